# Initial kernel scaffold; baseline (speedup 1.0000x reference)
import jax
import jax.numpy as jnp
from jax.experimental import pallas as pl


def kernel(verts, v_index, bary_weights):
    raise NotImplementedError("write your pallas kernel here")


# speedup vs baseline: 7.7159x; 7.7159x over previous
import functools

import jax
import jax.numpy as jnp
from jax import lax
from jax.experimental import pallas as pl
from jax.experimental.pallas import tpu as pltpu
from jax.experimental.pallas import tpu_sc as plsc

B = 32
V = 6890
H = 256
W = 256
P = H * W
F = B * 3
FP = 128
NW = 32
PW = P // NW
CH = 128
NCH = PW // CH
NLANE = 16
NJ = F // NLANE


def _uv_body(table, idx_a, idx_b, idx_c, bw_a, bw_b, bw_c, out,
             idx0, idx1, idx2, bw0, bw1, bw2,
             rows0, rows1, rows2, obuf, sem0, sem1, sem2):
    mesh_sc = lax.axis_index("c")
    mesh_sub = lax.axis_index("s")
    wid = mesh_sub * 2 + mesh_sc

    for t in range(NCH):
        base = wid * PW + t * CH
        sl = pl.ds(base, CH)
        pltpu.sync_copy(idx_a.at[sl], idx0)
        pltpu.sync_copy(idx_b.at[sl], idx1)
        pltpu.sync_copy(idx_c.at[sl], idx2)
        pltpu.sync_copy(bw_a.at[sl], bw0)
        pltpu.sync_copy(bw_b.at[sl], bw1)
        pltpu.sync_copy(bw_c.at[sl], bw2)
        c0 = pltpu.async_copy(table.at[idx0], rows0, sem0)
        c1 = pltpu.async_copy(table.at[idx1], rows1, sem1)
        c2 = pltpu.async_copy(table.at[idx2], rows2, sem2)
        c0.wait()
        c1.wait()
        c2.wait()

        def blend(g, carry):
            wv0 = bw0[pl.ds(g * NLANE, NLANE)]
            wv1 = bw1[pl.ds(g * NLANE, NLANE)]
            wv2 = bw2[pl.ds(g * NLANE, NLANE)]
            for u in range(NLANE):
                i = g * NLANE + u
                w0 = jnp.full((NLANE,), wv0[u], dtype=jnp.float32)
                w1 = jnp.full((NLANE,), wv1[u], dtype=jnp.float32)
                w2 = jnp.full((NLANE,), wv2[u], dtype=jnp.float32)
                for j in range(NJ):
                    js = pl.ds(j * NLANE, NLANE)
                    obuf[i, js] = (w0 * rows0[i, js] + w1 * rows1[i, js]
                                   + w2 * rows2[i, js])
            return carry

        lax.fori_loop(0, CH // NLANE, blend, 0)
        pltpu.sync_copy(obuf, out.at[sl])


@functools.partial(jax.jit, static_argnames=())
def _uv_sc(table, idx_a, idx_b, idx_c, bw_a, bw_b, bw_c):
    mesh = plsc.VectorSubcoreMesh(core_axis_name="c", subcore_axis_name="s")
    return pl.kernel(
        _uv_body,
        out_type=jax.ShapeDtypeStruct((P, F), jnp.float32),
        mesh=mesh,
        scratch_types=[
            pltpu.VMEM((CH,), jnp.int32),
            pltpu.VMEM((CH,), jnp.int32),
            pltpu.VMEM((CH,), jnp.int32),
            pltpu.VMEM((CH,), jnp.float32),
            pltpu.VMEM((CH,), jnp.float32),
            pltpu.VMEM((CH,), jnp.float32),
            pltpu.VMEM((CH, FP), jnp.float32),
            pltpu.VMEM((CH, FP), jnp.float32),
            pltpu.VMEM((CH, FP), jnp.float32),
            pltpu.VMEM((CH, F), jnp.float32),
            pltpu.SemaphoreType.DMA,
            pltpu.SemaphoreType.DMA,
            pltpu.SemaphoreType.DMA,
        ],
    )(table, idx_a, idx_b, idx_c, bw_a, bw_b, bw_c)


def kernel(verts, v_index, bary_weights):
    if verts.ndim == 2:
        verts = verts[None, ...]
    table = jnp.transpose(verts, (1, 0, 2)).reshape(V, F)
    table = jnp.pad(table, ((0, 0), (0, FP - F)))
    idx = v_index.reshape(P, 3).astype(jnp.int32)
    bw = bary_weights.reshape(P, 3)
    out_t = _uv_sc(table, idx[:, 0], idx[:, 1], idx[:, 2],
                   bw[:, 0], bw[:, 1], bw[:, 2])
    return jnp.transpose(out_t.reshape(P, B, 3), (1, 0, 2)).reshape(B, H, W, 3)


# speedup vs baseline: 11.6413x; 1.5087x over previous
import functools

import jax
import jax.numpy as jnp
from jax import lax
from jax.experimental import pallas as pl
from jax.experimental.pallas import tpu as pltpu
from jax.experimental.pallas import tpu_sc as plsc

B = 32
V = 6890
H = 256
W = 256
P = H * W
F = B * 3
FP = 128
NW = 32
PW = P // NW
CH = 64
NCH = PW // CH
NLANE = 16
NJ = F // NLANE


def _uv_body(table, idx_a, idx_b, idx_c, bw_a, bw_b, bw_c, out,
             idxw0, idxw1, idxw2, bww0, bww1, bww2,
             ra0, ra1, ra2, rb0, rb1, rb2, obuf, gsem0, gsem1, wbsem):
    wid = lax.axis_index("s") * 2 + lax.axis_index("c")
    wbase = wid * PW
    idxw = (idxw0, idxw1, idxw2)
    bww = (bww0, bww1, bww2)
    rows = ((ra0, ra1, ra2), (rb0, rb1, rb2))
    gsems = (gsem0, gsem1)

    stage = [
        pltpu.async_copy(idx_a.at[pl.ds(wbase, PW)], idxw0, wbsem),
        pltpu.async_copy(idx_b.at[pl.ds(wbase, PW)], idxw1, wbsem),
        pltpu.async_copy(idx_c.at[pl.ds(wbase, PW)], idxw2, wbsem),
        pltpu.async_copy(bw_a.at[pl.ds(wbase, PW)], bww0, wbsem),
        pltpu.async_copy(bw_b.at[pl.ds(wbase, PW)], bww1, wbsem),
        pltpu.async_copy(bw_c.at[pl.ds(wbase, PW)], bww2, wbsem),
    ]
    for c in stage:
        c.wait()

    def fire(t, slot):
        ts = pl.ds(t * CH, CH)
        for k in range(3):
            pltpu.async_copy(table.at[idxw[k].at[ts]], rows[slot][k],
                             gsems[slot])

    def drain_gather(slot):
        for k in range(3):
            pltpu.make_async_copy(table.at[pl.ds(0, CH)], rows[slot][k],
                                  gsems[slot]).wait()

    def drain_wb():
        pltpu.make_async_copy(obuf, out.at[pl.ds(wbase, CH)], wbsem).wait()

    def blend(t, slot):
        boff = t * CH
        r0, r1, r2 = rows[slot]

        def group(g, carry):
            wv0 = bww[0][pl.ds(boff + g * NLANE, NLANE)]
            wv1 = bww[1][pl.ds(boff + g * NLANE, NLANE)]
            wv2 = bww[2][pl.ds(boff + g * NLANE, NLANE)]
            for u in range(NLANE):
                i = g * NLANE + u
                w0 = jnp.full((NLANE,), wv0[u], dtype=jnp.float32)
                w1 = jnp.full((NLANE,), wv1[u], dtype=jnp.float32)
                w2 = jnp.full((NLANE,), wv2[u], dtype=jnp.float32)
                for j in range(NJ):
                    js = pl.ds(j * NLANE, NLANE)
                    obuf[i, js] = (w0 * r0[i, js] + w1 * r1[i, js]
                                   + w2 * r2[i, js])
            return carry

        lax.fori_loop(0, CH // NLANE, group, 0)

    fire(0, 0)

    def step(g, carry):
        t_a = 2 * g
        fire(t_a + 1, 1)
        drain_gather(0)

        @pl.when(g > 0)
        def _():
            drain_wb()

        blend(t_a, 0)
        pltpu.async_copy(obuf, out.at[pl.ds(wbase + t_a * CH, CH)], wbsem)

        @pl.when(g < NCH // 2 - 1)
        def _():
            fire(t_a + 2, 0)

        drain_gather(1)
        drain_wb()
        blend(t_a + 1, 1)
        pltpu.async_copy(obuf, out.at[pl.ds(wbase + (t_a + 1) * CH, CH)],
                         wbsem)
        return carry

    lax.fori_loop(0, NCH // 2, step, 0)
    drain_wb()


@functools.partial(jax.jit, static_argnames=())
def _uv_sc(table, idx_a, idx_b, idx_c, bw_a, bw_b, bw_c):
    mesh = plsc.VectorSubcoreMesh(core_axis_name="c", subcore_axis_name="s")
    return pl.kernel(
        _uv_body,
        out_type=jax.ShapeDtypeStruct((P, F), jnp.float32),
        mesh=mesh,
        scratch_types=[
            pltpu.VMEM((PW,), jnp.int32),
            pltpu.VMEM((PW,), jnp.int32),
            pltpu.VMEM((PW,), jnp.int32),
            pltpu.VMEM((PW,), jnp.float32),
            pltpu.VMEM((PW,), jnp.float32),
            pltpu.VMEM((PW,), jnp.float32),
            pltpu.VMEM((CH, FP), jnp.float32),
            pltpu.VMEM((CH, FP), jnp.float32),
            pltpu.VMEM((CH, FP), jnp.float32),
            pltpu.VMEM((CH, FP), jnp.float32),
            pltpu.VMEM((CH, FP), jnp.float32),
            pltpu.VMEM((CH, FP), jnp.float32),
            pltpu.VMEM((CH, F), jnp.float32),
            pltpu.SemaphoreType.DMA,
            pltpu.SemaphoreType.DMA,
            pltpu.SemaphoreType.DMA,
        ],
    )(table, idx_a, idx_b, idx_c, bw_a, bw_b, bw_c)


def kernel(verts, v_index, bary_weights):
    if verts.ndim == 2:
        verts = verts[None, ...]
    table = jnp.transpose(verts, (1, 0, 2)).reshape(V, F)
    table = jnp.pad(table, ((0, 0), (0, FP - F)))
    idx = v_index.reshape(P, 3).astype(jnp.int32)
    bw = bary_weights.reshape(P, 3)
    out_t = _uv_sc(table, idx[:, 0], idx[:, 1], idx[:, 2],
                   bw[:, 0], bw[:, 1], bw[:, 2])
    return jnp.transpose(out_t.reshape(P, B, 3), (1, 0, 2)).reshape(B, H, W, 3)
